# Initial kernel scaffold; baseline (speedup 1.0000x reference)
import jax
import jax.numpy as jnp
from jax.experimental import pallas as pl


def kernel(logits):
    raise NotImplementedError("write your pallas kernel here")


# speedup vs baseline: 4.8026x; 4.8026x over previous
import jax
import jax.numpy as jnp
from jax.experimental import pallas as pl
from jax.experimental.pallas import tpu as pltpu

_TEMP = 0.7
_K = 50
_B = 128
_V = 100000
_RB = 8
_SEL = 64


def _main_body(x_ref, masked_ref, selidx_ref, selval_ref, u_scr):
    x = x_ref[...]
    y = x / jnp.float32(_TEMP)
    ymax = jnp.max(y, axis=1, keepdims=True)
    u = jnp.exp(y - ymax)
    s = jnp.sum(u, axis=1, keepdims=True)
    col = jax.lax.broadcasted_iota(jnp.int32, (_RB, _V), 1)
    col64 = jax.lax.broadcasted_iota(jnp.int32, (_RB, _SEL), 1)
    u_scr[...] = u
    vals = u[:, :_SEL] / s
    masked_ref[...] = jnp.zeros((_RB, _V), jnp.float32)
    selidx_ref[...] = jnp.zeros((_RB, _SEL), jnp.int32)
    selval_ref[...] = jnp.zeros((_RB, _SEL), jnp.float32)

    def body(j, carry):
        uu = u_scr[...]
        m = jnp.min(uu, axis=1, keepdims=True)
        idx = jnp.min(jnp.where(uu == m, col, jnp.int32(2**30)),
                      axis=1, keepdims=True)
        u_scr[...] = jnp.where(col == idx, jnp.float32(jnp.inf), uu)
        valj = jnp.sum(jnp.where(col64 == j, vals, 0.0),
                       axis=1, keepdims=True)
        masked_ref[...] = jnp.where(col == idx, valj, masked_ref[...])
        selidx_ref[...] = jnp.where(col64 == j, idx, selidx_ref[...])
        selval_ref[...] = jnp.where(col64 == j, valj, selval_ref[...])
        return carry

    jax.lax.fori_loop(0, _K, body, 0)
    masked_ref[...] = jnp.where(col == 0, 0.0, masked_ref[...])


def _rotl(x, d):
    return (x << jnp.uint32(d)) | (x >> jnp.uint32(32 - d))


def _sample_body(selidx_ref, selval_ref, out_ref):
    sel = selidx_ref[...]
    val = selval_ref[...]
    col = jax.lax.broadcasted_iota(jnp.int32, (_B, _SEL), 1)
    row = jax.lax.broadcasted_iota(jnp.int32, (_B, _SEL), 0)
    valid = (col < _K) & (sel != 0)
    x1 = (row * _V + sel).astype(jnp.uint32)
    x0 = jnp.zeros((_B, _SEL), jnp.uint32)
    ks0, ks1 = jnp.uint32(0), jnp.uint32(1)
    ks2 = jnp.uint32(0x1BD11BDA) ^ ks0 ^ ks1
    ks = (ks0, ks1, ks2)
    x0 = x0 + ks[0]
    x1 = x1 + ks[1]
    rotations = ((13, 15, 26, 6), (17, 29, 16, 24))
    for i in range(5):
        for r in rotations[i % 2]:
            x0 = x0 + x1
            x1 = _rotl(x1, r) ^ x0
        x0 = x0 + ks[(i + 1) % 3]
        x1 = x1 + ks[(i + 2) % 3] + jnp.uint32(i + 1)
    bits = x0 ^ x1
    fb = (bits >> jnp.uint32(9)) | jnp.uint32(0x3F800000)
    tiny = jnp.float32(jnp.finfo(jnp.float32).tiny)
    fl = jax.lax.bitcast_convert_type(fb, jnp.float32) - 1.0
    un = jnp.maximum(tiny, fl * (jnp.float32(1.0) - tiny) + tiny)
    g = -jnp.log(-jnp.log(un))
    score = jnp.where(valid, jnp.log(jnp.maximum(val, 1e-30)) + g,
                      -jnp.float32(jnp.inf))
    best = jnp.max(score, axis=1, keepdims=True)
    samp = jnp.min(jnp.where(score == best, sel, jnp.int32(_V + 1)),
                   axis=1, keepdims=True)
    out_ref[...] = samp


def kernel(logits):
    nblk = _B // _RB
    masked, selidx, selval = pl.pallas_call(
        _main_body,
        grid=(nblk,),
        in_specs=[pl.BlockSpec((_RB, _V), lambda i: (i, 0))],
        out_specs=[
            pl.BlockSpec((_RB, _V), lambda i: (i, 0)),
            pl.BlockSpec((_RB, _SEL), lambda i: (i, 0)),
            pl.BlockSpec((_RB, _SEL), lambda i: (i, 0)),
        ],
        out_shape=[
            jax.ShapeDtypeStruct((_B, _V), jnp.float32),
            jax.ShapeDtypeStruct((_B, _SEL), jnp.int32),
            jax.ShapeDtypeStruct((_B, _SEL), jnp.float32),
        ],
        scratch_shapes=[pltpu.VMEM((_RB, _V), jnp.float32)],
    )(logits)

    sample = pl.pallas_call(
        _sample_body,
        out_shape=jax.ShapeDtypeStruct((_B, 1), jnp.int32),
    )(selidx, selval)
    return masked, sample.astype(jnp.int64)


# speedup vs baseline: 16.5559x; 3.4473x over previous
import functools

import jax
import jax.numpy as jnp
from jax import lax
from jax.experimental import pallas as pl
from jax.experimental.pallas import tpu as pltpu
from jax.experimental.pallas import tpu_sc as plsc

_TEMP = 0.7
_K = 50
_B = 128
_V = 100000
_SEGW = 256
_NSEG = 416
_VP = 106496
_CHUNK = 8192
_NCHUNK = 13
_RB = 8
_NSEL = 64
_CAP = 256
_RPT = 4
_SEL = 64


def _k1a_body(x_ref, ymax_ref):
    c = pl.program_id(1)
    x = x_ref[...]
    col = c * _CHUNK + lax.broadcasted_iota(jnp.int32, (_RB, _CHUNK), 1)
    y = jnp.where(col < _V, x / jnp.float32(_TEMP), -jnp.float32(jnp.inf))
    mb = jnp.broadcast_to(jnp.max(y, axis=1, keepdims=True), (_RB, 128))

    @pl.when(c == 0)
    def _():
        ymax_ref[...] = mb

    @pl.when(c != 0)
    def _():
        ymax_ref[...] = jnp.maximum(ymax_ref[...], mb)


def _k1b_body(x_ref, ymax_ref, u_ref, segmin_ref, s_ref):
    c = pl.program_id(1)
    x = x_ref[...]
    ym = ymax_ref[...][:, :1]
    col = c * _CHUNK + lax.broadcasted_iota(jnp.int32, (_RB, _CHUNK), 1)
    valid = col < _V
    u = jnp.exp(x / jnp.float32(_TEMP) - ym)
    u_out = jnp.where(valid, u, jnp.float32(jnp.inf))
    u_ref[...] = u_out
    s_part = jnp.sum(jnp.where(valid, u, 0.0), axis=1, keepdims=True)
    nsegs = _CHUNK // _SEGW
    lane416 = lax.broadcasted_iota(jnp.int32, (_RB, _NSEG), 1)
    segs = jnp.full((_RB, _NSEG), jnp.inf, jnp.float32)
    for t in range(nsegs):
        mt = jnp.min(u_out[:, t * _SEGW:(t + 1) * _SEGW], axis=1,
                     keepdims=True)
        segs = jnp.where(lane416 == c * nsegs + t,
                         jnp.broadcast_to(mt, (_RB, _NSEG)), segs)
    in_range = (lane416 >= c * nsegs) & (lane416 < (c + 1) * nsegs)
    sb = jnp.broadcast_to(s_part, (_RB, 128))

    @pl.when(c == 0)
    def _():
        segmin_ref[...] = segs
        s_ref[...] = sb

    @pl.when(c != 0)
    def _():
        segmin_ref[...] = jnp.where(in_range, segs, segmin_ref[...])
        s_ref[...] = s_ref[...] + sb


def _bi(s):
    return jnp.full((16,), s, jnp.int32)


def _bf(s):
    return jnp.full((16,), s, jnp.float32)


_mesh = plsc.VectorSubcoreMesh(core_axis_name="c", subcore_axis_name="s")


@functools.partial(
    pl.kernel,
    mesh=_mesh,
    compiler_params=pltpu.CompilerParams(needs_layout_passes=False),
    out_type=[
        jax.ShapeDtypeStruct((_B, _V), jnp.float32),
        jax.ShapeDtypeStruct((_B, _SEL), jnp.int32),
        jax.ShapeDtypeStruct((_B, _SEL), jnp.float32),
    ],
    scratch_types=[
        pltpu.VMEM((_NSEG,), jnp.float32),
        pltpu.VMEM((_NSEL,), jnp.int32),
        pltpu.VMEM((_NSEL,), jnp.int32),
        pltpu.VMEM((_NSEL, _SEGW), jnp.float32),
        pltpu.VMEM((_CAP,), jnp.float32),
        pltpu.VMEM((_CAP,), jnp.int32),
        pltpu.VMEM((_SEGW,), jnp.float32),
        pltpu.VMEM((16,), jnp.float32),
        pltpu.VMEM((_V,), jnp.float32),
        pltpu.VMEM((_SEL,), jnp.int32),
        pltpu.VMEM((_SEL,), jnp.float32),
        pltpu.SemaphoreType.DMA,
    ],
)
def _sc_select(u2_h, segmin_h, spre_h, masked_h, selidx_h, selval_h,
               segmin_v, segid_v, segbase_v, segbuf, cand_u, cand_i,
               prefix_v, s_v, rowbuf, selidx_v, selval_v, sem):
    wid = lax.axis_index("s") * 2 + lax.axis_index("c")
    lane = lax.broadcasted_iota(jnp.int32, (16,), 0)
    inf_v = jnp.full((16,), jnp.inf, jnp.float32)
    big_v = jnp.full((16,), 2**30, jnp.int32)
    zero_v = jnp.zeros((16,), jnp.float32)

    def zbody(i, carry):
        rowbuf[pl.ds(i * 16, 16)] = zero_v
        return carry

    lax.fori_loop(0, _V // 16, zbody, 0)

    def row_body(rr, carry0):
        row = wid * _RPT + rr
        pltpu.sync_copy(segmin_h.at[row], segmin_v)
        pltpu.sync_copy(spre_h.at[row], s_v)
        pltpu.sync_copy(u2_h.at[row * _NSEG], prefix_v)

        nv = _NSEG // 16

        def ext_body(j, st):
            a0, a1, a2, a3, tau = st
            mval = inf_v
            for k in range(nv):
                mval = jnp.minimum(mval, segmin_v[pl.ds(k * 16, 16)])
            mb = _bf(jnp.min(mval))
            mid = big_v
            for k in range(nv):
                v = segmin_v[pl.ds(k * 16, 16)]
                mid = jnp.minimum(
                    mid, jnp.where(v == mb, lane + k * 16, big_v))
            sb = _bi(jnp.min(mid))
            plsc.store_scatter(segmin_v, [sb], inf_v, mask=lane == 0)
            g = j // 16
            l = j - g * 16
            le = lane == _bi(l)
            gv = _bi(g)
            a0 = jnp.where(le & (gv == 0), sb, a0)
            a1 = jnp.where(le & (gv == 1), sb, a1)
            a2 = jnp.where(le & (gv == 2), sb, a2)
            a3 = jnp.where(le & (gv == 3), sb, a3)
            tau = jnp.where(_bi(j) == _K - 1, mb, tau)
            return (a0, a1, a2, a3, tau)

        a0, a1, a2, a3, tau = lax.fori_loop(
            0, _NSEL, ext_body, (big_v, big_v, big_v, big_v, inf_v))

        rowbase = _bi(row * _NSEG)
        avs = (a0, a1, a2, a3)
        for g in range(4):
            segid_v[pl.ds(g * 16, 16)] = avs[g] + rowbase
            segbase_v[pl.ds(g * 16, 16)] = avs[g] * _SEGW
        pltpu.async_copy(u2_h.at[segid_v], segbuf, sem).wait()

        for k in range(_CAP // 16):
            cand_u[pl.ds(k * 16, 16)] = inf_v
            cand_i[pl.ds(k * 16, 16)] = big_v

        def seg_body(s, cnt_v):
            base = plsc.load_gather(segbase_v, [_bi(s)])
            for o in range(_SEGW // 16):
                u_v = segbuf[s, pl.ds(o * 16, 16)]
                m = u_v <= tau
                pc = plsc.all_reduce_population_count(m)
                mi = m.astype(jnp.int32)
                pos = cnt_v + plsc.cumsum(mi) - mi
                wm = m & (pos < _CAP)
                idx_v = base + (o * 16 + lane)
                plsc.store_scatter(cand_u, [pos], u_v, mask=wm)
                plsc.store_scatter(cand_i, [pos], idx_v, mask=wm)
                cnt_v = cnt_v + pc
            return cnt_v

        lax.fori_loop(0, _NSEL, seg_body, jnp.zeros((16,), jnp.int32))

        def fin_body(j, st):
            s0, s1, s2, s3 = st
            mval = inf_v
            for k in range(_CAP // 16):
                mval = jnp.minimum(mval, cand_u[pl.ds(k * 16, 16)])
            mb = _bf(jnp.min(mval))
            mid = big_v
            for k in range(_CAP // 16):
                uv = cand_u[pl.ds(k * 16, 16)]
                iv = cand_i[pl.ds(k * 16, 16)]
                mid = jnp.minimum(mid, jnp.where(uv == mb, iv, big_v))
            ib = _bi(jnp.min(mid))
            for k in range(_CAP // 16):
                uv = cand_u[pl.ds(k * 16, 16)]
                iv = cand_i[pl.ds(k * 16, 16)]
                cand_u[pl.ds(k * 16, 16)] = jnp.where(iv == ib, inf_v, uv)
            g = j // 16
            l = j - g * 16
            le = lane == _bi(l)
            gv = _bi(g)
            s0 = jnp.where(le & (gv == 0), ib, s0)
            s1 = jnp.where(le & (gv == 1), ib, s1)
            s2 = jnp.where(le & (gv == 2), ib, s2)
            s3 = jnp.where(le & (gv == 3), ib, s3)
            return (s0, s1, s2, s3)

        sel = lax.fori_loop(0, _K, fin_body, (big_v, big_v, big_v, big_v))

        sv = s_v[pl.ds(0, 16)]
        for g in range(4):
            selidx_v[pl.ds(g * 16, 16)] = sel[g]
            selval_v[pl.ds(g * 16, 16)] = prefix_v[pl.ds(g * 16, 16)] / sv
        for g in range(4):
            valid = (sel[g] < _bi(_V)) & ((lane + g * 16) < _K)
            pv = selval_v[pl.ds(g * 16, 16)]
            plsc.store_scatter(rowbuf, [sel[g]], pv, mask=valid)
        w0 = rowbuf[pl.ds(0, 16)]
        rowbuf[pl.ds(0, 16)] = jnp.where(lane == 0, zero_v, w0)
        pltpu.sync_copy(rowbuf, masked_h.at[row])
        for g in range(4):
            valid = (sel[g] < _bi(_V)) & ((lane + g * 16) < _K)
            plsc.store_scatter(rowbuf, [sel[g]], zero_v, mask=valid)
        pltpu.sync_copy(selidx_v, selidx_h.at[row])
        pltpu.sync_copy(selval_v, selval_h.at[row])
        return carry0

    lax.fori_loop(0, _RPT, row_body, 0)


def _rotl(x, d):
    return (x << jnp.uint32(d)) | (x >> jnp.uint32(32 - d))


def _sample_body(selidx_ref, selval_ref, out_ref):
    sel = selidx_ref[...]
    val = selval_ref[...]
    col = lax.broadcasted_iota(jnp.int32, (_B, _SEL), 1)
    row = lax.broadcasted_iota(jnp.int32, (_B, _SEL), 0)
    valid = (col < _K) & (sel != 0) & (sel < _V)
    x1 = (row * _V + sel).astype(jnp.uint32)
    x0 = jnp.zeros((_B, _SEL), jnp.uint32)
    ks = (jnp.uint32(0), jnp.uint32(1),
          jnp.uint32(0x1BD11BDA) ^ jnp.uint32(0) ^ jnp.uint32(1))
    x0 = x0 + ks[0]
    x1 = x1 + ks[1]
    rotations = ((13, 15, 26, 6), (17, 29, 16, 24))
    for i in range(5):
        for r in rotations[i % 2]:
            x0 = x0 + x1
            x1 = _rotl(x1, r) ^ x0
        x0 = x0 + ks[(i + 1) % 3]
        x1 = x1 + ks[(i + 2) % 3] + jnp.uint32(i + 1)
    bits = x0 ^ x1
    fb = (bits >> jnp.uint32(9)) | jnp.uint32(0x3F800000)
    tiny = jnp.float32(jnp.finfo(jnp.float32).tiny)
    fl = lax.bitcast_convert_type(fb, jnp.float32) - 1.0
    un = jnp.maximum(tiny, fl * (jnp.float32(1.0) - tiny) + tiny)
    g = -jnp.log(-jnp.log(un))
    score = jnp.where(valid, jnp.log(jnp.maximum(val, 1e-30)) + g,
                      -jnp.float32(jnp.inf))
    best = jnp.max(score, axis=1, keepdims=True)
    samp = jnp.min(jnp.where(score == best, sel, jnp.int32(_V + 1)),
                   axis=1, keepdims=True)
    out_ref[...] = samp


def kernel(logits):
    nblk = _B // _RB
    ymax = pl.pallas_call(
        _k1a_body,
        grid=(nblk, _NCHUNK),
        in_specs=[pl.BlockSpec((_RB, _CHUNK), lambda i, c: (i, c))],
        out_specs=pl.BlockSpec((_RB, 128), lambda i, c: (i, 0)),
        out_shape=jax.ShapeDtypeStruct((_B, 128), jnp.float32),
    )(logits)

    u, segmin, s = pl.pallas_call(
        _k1b_body,
        grid=(nblk, _NCHUNK),
        in_specs=[
            pl.BlockSpec((_RB, _CHUNK), lambda i, c: (i, c)),
            pl.BlockSpec((_RB, 128), lambda i, c: (i, 0)),
        ],
        out_specs=[
            pl.BlockSpec((_RB, _CHUNK), lambda i, c: (i, c)),
            pl.BlockSpec((_RB, _NSEG), lambda i, c: (i, 0)),
            pl.BlockSpec((_RB, 128), lambda i, c: (i, 0)),
        ],
        out_shape=[
            jax.ShapeDtypeStruct((_B, _VP), jnp.float32),
            jax.ShapeDtypeStruct((_B, _NSEG), jnp.float32),
            jax.ShapeDtypeStruct((_B, 128), jnp.float32),
        ],
    )(logits, ymax)

    u2 = u.reshape(_B * _NSEG, _SEGW)
    spre = s[:, :16]
    masked, selidx, selval = _sc_select(u2, segmin, spre)

    sample = pl.pallas_call(
        _sample_body,
        out_shape=jax.ShapeDtypeStruct((_B, 1), jnp.int32),
    )(selidx, selval)
    return masked, sample.astype(jnp.int64)


# speedup vs baseline: 18.9800x; 1.1464x over previous
import functools

import jax
import jax.numpy as jnp
from jax import lax
from jax.experimental import pallas as pl
from jax.experimental.pallas import tpu as pltpu
from jax.experimental.pallas import tpu_sc as plsc

_TEMP = 0.7
_K = 50
_B = 128
_V = 100000
_SEGW = 256
_NSEG = 416
_VP = 106496
_CHUNK = 8192
_NCHUNK = 13
_RB = 8
_NSEL = 64
_CAP = 256
_RPT = 4
_SEL = 64


def _k1a_body(x_ref, ymax_ref):
    c = pl.program_id(1)
    x = x_ref[...]
    col = c * _CHUNK + lax.broadcasted_iota(jnp.int32, (_RB, _CHUNK), 1)
    y = jnp.where(col < _V, x / jnp.float32(_TEMP), -jnp.float32(jnp.inf))
    mb = jnp.broadcast_to(jnp.max(y, axis=1, keepdims=True), (_RB, 128))

    @pl.when(c == 0)
    def _():
        ymax_ref[...] = mb

    @pl.when(c != 0)
    def _():
        ymax_ref[...] = jnp.maximum(ymax_ref[...], mb)


def _k1b_body(x_ref, ymax_ref, u_ref, segmin_ref, s_ref):
    c = pl.program_id(1)
    x = x_ref[...]
    ym = ymax_ref[...][:, :1]
    col = c * _CHUNK + lax.broadcasted_iota(jnp.int32, (_RB, _CHUNK), 1)
    valid = col < _V
    u = jnp.exp(x / jnp.float32(_TEMP) - ym)
    u_out = jnp.where(valid, u, jnp.float32(jnp.inf))
    u_ref[...] = u_out
    s_part = jnp.sum(jnp.where(valid, u, 0.0), axis=1, keepdims=True)
    nsegs = _CHUNK // _SEGW
    lane416 = lax.broadcasted_iota(jnp.int32, (_RB, _NSEG), 1)
    segs = jnp.full((_RB, _NSEG), jnp.inf, jnp.float32)
    for t in range(nsegs):
        mt = jnp.min(u_out[:, t * _SEGW:(t + 1) * _SEGW], axis=1,
                     keepdims=True)
        segs = jnp.where(lane416 == c * nsegs + t,
                         jnp.broadcast_to(mt, (_RB, _NSEG)), segs)
    in_range = (lane416 >= c * nsegs) & (lane416 < (c + 1) * nsegs)
    sb = jnp.broadcast_to(s_part, (_RB, 128))

    @pl.when(c == 0)
    def _():
        segmin_ref[...] = segs
        s_ref[...] = sb

    @pl.when(c != 0)
    def _():
        segmin_ref[...] = jnp.where(in_range, segs, segmin_ref[...])
        s_ref[...] = s_ref[...] + sb


def _bi(s):
    return jnp.full((16,), s, jnp.int32)


def _bf(s):
    return jnp.full((16,), s, jnp.float32)


_mesh = plsc.VectorSubcoreMesh(core_axis_name="c", subcore_axis_name="s")


@functools.partial(
    pl.kernel,
    mesh=_mesh,
    compiler_params=pltpu.CompilerParams(needs_layout_passes=False),
    out_type=[
        jax.ShapeDtypeStruct((_B, _V), jnp.float32),
        jax.ShapeDtypeStruct((_B, _SEL), jnp.int32),
        jax.ShapeDtypeStruct((_B, _SEL), jnp.float32),
    ],
    scratch_types=[
        pltpu.VMEM((_NSEG,), jnp.float32),
        pltpu.VMEM((_NSEL,), jnp.int32),
        pltpu.VMEM((_NSEL,), jnp.int32),
        pltpu.VMEM((_NSEL, _SEGW), jnp.float32),
        pltpu.VMEM((_CAP,), jnp.float32),
        pltpu.VMEM((_CAP,), jnp.int32),
        pltpu.VMEM((_SEGW,), jnp.float32),
        pltpu.VMEM((16,), jnp.float32),
        pltpu.VMEM((_V,), jnp.float32),
        pltpu.VMEM((_SEL,), jnp.int32),
        pltpu.VMEM((_SEL,), jnp.float32),
        pltpu.SemaphoreType.DMA,
    ],
)
def _sc_select(u_h, segmin_h, spre_h, masked_h, selidx_h, selval_h,
               segmin_v, segid_v, segbase_v, segbuf, cand_u, cand_i,
               prefix_v, s_v, rowbuf, selidx_v, selval_v, sem):
    wid = lax.axis_index("s") * 2 + lax.axis_index("c")
    lane = lax.broadcasted_iota(jnp.int32, (16,), 0)
    inf_v = jnp.full((16,), jnp.inf, jnp.float32)
    big_v = jnp.full((16,), 2**30, jnp.int32)
    zero_v = jnp.zeros((16,), jnp.float32)

    def zbody(i, carry):
        for t in range(8):
            rowbuf[pl.ds(i * 128 + t * 16, 16)] = zero_v
        return carry

    lax.fori_loop(0, _V // 128, zbody, 0)
    for t in range(_V // 128 * 128, _V, 16):
        rowbuf[pl.ds(t, 16)] = zero_v

    def row_body(rr, carry0):
        row = wid * _RPT + rr
        pltpu.sync_copy(segmin_h.at[row], segmin_v)
        pltpu.sync_copy(spre_h.at[row], s_v)
        pltpu.sync_copy(u_h.at[row, pl.ds(0, _SEGW)], prefix_v)

        nv = _NSEG // 16

        def ext_body(j, st):
            a0, a1, a2, a3, tau = st
            mval = inf_v
            for k in range(nv):
                mval = jnp.minimum(mval, segmin_v[pl.ds(k * 16, 16)])
            mb = _bf(jnp.min(mval))
            mid = big_v
            for k in range(nv):
                v = segmin_v[pl.ds(k * 16, 16)]
                mid = jnp.minimum(
                    mid, jnp.where(v == mb, lane + k * 16, big_v))
            sid = jnp.min(mid)
            sb = _bi(sid)
            pltpu.async_copy(
                u_h.at[row, pl.ds(sid * _SEGW, _SEGW)], segbuf.at[j], sem)
            plsc.store_scatter(segmin_v, [sb], inf_v, mask=lane == 0)
            g = j // 16
            l = j - g * 16
            le = lane == _bi(l)
            gv = _bi(g)
            a0 = jnp.where(le & (gv == 0), sb, a0)
            a1 = jnp.where(le & (gv == 1), sb, a1)
            a2 = jnp.where(le & (gv == 2), sb, a2)
            a3 = jnp.where(le & (gv == 3), sb, a3)
            tau = jnp.where(_bi(j) == _K - 1, mb, tau)
            return (a0, a1, a2, a3, tau)

        a0, a1, a2, a3, tau = lax.fori_loop(
            0, _NSEL, ext_body, (big_v, big_v, big_v, big_v, inf_v))

        avs = (a0, a1, a2, a3)
        for g in range(4):
            segbase_v[pl.ds(g * 16, 16)] = avs[g] * _SEGW
        pltpu.make_async_copy(
            u_h.at[pl.ds(0, _NSEL), pl.ds(0, _SEGW)], segbuf, sem).wait()

        for k in range(_CAP // 16):
            cand_u[pl.ds(k * 16, 16)] = inf_v
            cand_i[pl.ds(k * 16, 16)] = big_v

        def seg_body(s, cnt_v):
            base = plsc.load_gather(segbase_v, [_bi(s)])
            for o in range(_SEGW // 16):
                u_v = segbuf[s, pl.ds(o * 16, 16)]
                m = u_v <= tau
                pc = plsc.all_reduce_population_count(m)
                mi = m.astype(jnp.int32)
                pos = cnt_v + plsc.cumsum(mi) - mi
                wm = m & (pos < _CAP)
                idx_v = base + (o * 16 + lane)
                plsc.store_scatter(cand_u, [pos], u_v, mask=wm)
                plsc.store_scatter(cand_i, [pos], idx_v, mask=wm)
                cnt_v = cnt_v + pc
            return cnt_v

        lax.fori_loop(0, _NSEL, seg_body, jnp.zeros((16,), jnp.int32))

        def fin_body(j, st):
            s0, s1, s2, s3 = st
            mval = inf_v
            for k in range(_CAP // 16):
                mval = jnp.minimum(mval, cand_u[pl.ds(k * 16, 16)])
            mb = _bf(jnp.min(mval))
            mid = big_v
            for k in range(_CAP // 16):
                uv = cand_u[pl.ds(k * 16, 16)]
                iv = cand_i[pl.ds(k * 16, 16)]
                mid = jnp.minimum(mid, jnp.where(uv == mb, iv, big_v))
            ib = _bi(jnp.min(mid))
            for k in range(_CAP // 16):
                uv = cand_u[pl.ds(k * 16, 16)]
                iv = cand_i[pl.ds(k * 16, 16)]
                cand_u[pl.ds(k * 16, 16)] = jnp.where(iv == ib, inf_v, uv)
            g = j // 16
            l = j - g * 16
            le = lane == _bi(l)
            gv = _bi(g)
            s0 = jnp.where(le & (gv == 0), ib, s0)
            s1 = jnp.where(le & (gv == 1), ib, s1)
            s2 = jnp.where(le & (gv == 2), ib, s2)
            s3 = jnp.where(le & (gv == 3), ib, s3)
            return (s0, s1, s2, s3)

        sel = lax.fori_loop(0, _K, fin_body, (big_v, big_v, big_v, big_v))

        sv = s_v[pl.ds(0, 16)]
        for g in range(4):
            selidx_v[pl.ds(g * 16, 16)] = sel[g]
            selval_v[pl.ds(g * 16, 16)] = prefix_v[pl.ds(g * 16, 16)] / sv
        for g in range(4):
            valid = (sel[g] < _bi(_V)) & ((lane + g * 16) < _K)
            pv = selval_v[pl.ds(g * 16, 16)]
            plsc.store_scatter(rowbuf, [sel[g]], pv, mask=valid)
        w0 = rowbuf[pl.ds(0, 16)]
        rowbuf[pl.ds(0, 16)] = jnp.where(lane == 0, zero_v, w0)
        pltpu.sync_copy(rowbuf, masked_h.at[row])
        for g in range(4):
            valid = (sel[g] < _bi(_V)) & ((lane + g * 16) < _K)
            plsc.store_scatter(rowbuf, [sel[g]], zero_v, mask=valid)
        pltpu.sync_copy(selidx_v, selidx_h.at[row])
        pltpu.sync_copy(selval_v, selval_h.at[row])
        return carry0

    lax.fori_loop(0, _RPT, row_body, 0)


def _rotl(x, d):
    return (x << jnp.uint32(d)) | (x >> jnp.uint32(32 - d))


def _sample_body(selidx_ref, selval_ref, out_ref):
    sel = selidx_ref[...]
    val = selval_ref[...]
    col = lax.broadcasted_iota(jnp.int32, (_B, _SEL), 1)
    row = lax.broadcasted_iota(jnp.int32, (_B, _SEL), 0)
    valid = (col < _K) & (sel != 0) & (sel < _V)
    x1 = (row * _V + sel).astype(jnp.uint32)
    x0 = jnp.zeros((_B, _SEL), jnp.uint32)
    ks = (jnp.uint32(0), jnp.uint32(1),
          jnp.uint32(0x1BD11BDA) ^ jnp.uint32(0) ^ jnp.uint32(1))
    x0 = x0 + ks[0]
    x1 = x1 + ks[1]
    rotations = ((13, 15, 26, 6), (17, 29, 16, 24))
    for i in range(5):
        for r in rotations[i % 2]:
            x0 = x0 + x1
            x1 = _rotl(x1, r) ^ x0
        x0 = x0 + ks[(i + 1) % 3]
        x1 = x1 + ks[(i + 2) % 3] + jnp.uint32(i + 1)
    bits = x0 ^ x1
    fb = (bits >> jnp.uint32(9)) | jnp.uint32(0x3F800000)
    tiny = jnp.float32(jnp.finfo(jnp.float32).tiny)
    fl = lax.bitcast_convert_type(fb, jnp.float32) - 1.0
    un = jnp.maximum(tiny, fl * (jnp.float32(1.0) - tiny) + tiny)
    g = -jnp.log(-jnp.log(un))
    score = jnp.where(valid, jnp.log(jnp.maximum(val, 1e-30)) + g,
                      -jnp.float32(jnp.inf))
    best = jnp.max(score, axis=1, keepdims=True)
    samp = jnp.min(jnp.where(score == best, sel, jnp.int32(_V + 1)),
                   axis=1, keepdims=True)
    out_ref[...] = samp


def kernel(logits):
    nblk = _B // _RB
    ymax = pl.pallas_call(
        _k1a_body,
        grid=(nblk, _NCHUNK),
        in_specs=[pl.BlockSpec((_RB, _CHUNK), lambda i, c: (i, c))],
        out_specs=pl.BlockSpec((_RB, 128), lambda i, c: (i, 0)),
        out_shape=jax.ShapeDtypeStruct((_B, 128), jnp.float32),
    )(logits)

    u, segmin, s = pl.pallas_call(
        _k1b_body,
        grid=(nblk, _NCHUNK),
        in_specs=[
            pl.BlockSpec((_RB, _CHUNK), lambda i, c: (i, c)),
            pl.BlockSpec((_RB, 128), lambda i, c: (i, 0)),
        ],
        out_specs=[
            pl.BlockSpec((_RB, _CHUNK), lambda i, c: (i, c)),
            pl.BlockSpec((_RB, _NSEG), lambda i, c: (i, 0)),
            pl.BlockSpec((_RB, 128), lambda i, c: (i, 0)),
        ],
        out_shape=[
            jax.ShapeDtypeStruct((_B, _VP), jnp.float32),
            jax.ShapeDtypeStruct((_B, _NSEG), jnp.float32),
            jax.ShapeDtypeStruct((_B, 128), jnp.float32),
        ],
    )(logits, ymax)

    spre = s[:, :16]
    masked, selidx, selval = _sc_select(u, segmin, spre)

    sample = pl.pallas_call(
        _sample_body,
        out_shape=jax.ShapeDtypeStruct((_B, 1), jnp.int32),
    )(selidx, selval)
    return masked, sample.astype(jnp.int64)


# speedup vs baseline: 24.8201x; 1.3077x over previous
import functools

import jax
import jax.numpy as jnp
from jax import lax
from jax.experimental import pallas as pl
from jax.experimental.pallas import tpu as pltpu
from jax.experimental.pallas import tpu_sc as plsc

_TEMP = 0.7
_K = 50
_B = 128
_V = 100000
_SEGW = 256
_NSEG = 448
_VP = 114688
_CHUNK = 16384
_NCHUNK = 7
_CHUNKA = 32768
_NCHUNKA = 4
_RB = 8
_NSEL = 64
_CAP = 256
_RPT = 4
_SEL = 64


def _k1a_body(x_ref, ymax_ref):
    c = pl.program_id(1)
    x = x_ref[...]
    col = c * _CHUNKA + lax.broadcasted_iota(jnp.int32, (_RB, _CHUNKA), 1)
    y = jnp.where(col < _V, x / jnp.float32(_TEMP), -jnp.float32(jnp.inf))
    mb = jnp.broadcast_to(jnp.max(y, axis=1, keepdims=True), (_RB, 128))

    @pl.when(c == 0)
    def _():
        ymax_ref[...] = mb

    @pl.when(c != 0)
    def _():
        ymax_ref[...] = jnp.maximum(ymax_ref[...], mb)


def _k1b_body(x_ref, ymax_ref, u_ref, segmin_ref, s_ref):
    c = pl.program_id(1)
    x = x_ref[...]
    ym = ymax_ref[...][:, :1]
    col = c * _CHUNK + lax.broadcasted_iota(jnp.int32, (_RB, _CHUNK), 1)
    valid = col < _V
    u = jnp.exp(x / jnp.float32(_TEMP) - ym)
    u_out = jnp.where(valid, u, jnp.float32(jnp.inf))
    u_ref[...] = u_out
    s_part = jnp.sum(jnp.where(valid, u, 0.0), axis=1, keepdims=True)
    nsegs = _CHUNK // _SEGW
    lane416 = lax.broadcasted_iota(jnp.int32, (_RB, _NSEG), 1)
    segs = jnp.full((_RB, _NSEG), jnp.inf, jnp.float32)
    for t in range(nsegs):
        mt = jnp.min(u_out[:, t * _SEGW:(t + 1) * _SEGW], axis=1,
                     keepdims=True)
        segs = jnp.where(lane416 == c * nsegs + t,
                         jnp.broadcast_to(mt, (_RB, _NSEG)), segs)
    in_range = (lane416 >= c * nsegs) & (lane416 < (c + 1) * nsegs)
    sb = jnp.broadcast_to(s_part, (_RB, 128))

    @pl.when(c == 0)
    def _():
        segmin_ref[...] = segs
        s_ref[...] = sb

    @pl.when(c != 0)
    def _():
        segmin_ref[...] = jnp.where(in_range, segs, segmin_ref[...])
        s_ref[...] = s_ref[...] + sb


def _bi(s):
    return jnp.full((16,), s, jnp.int32)


def _bf(s):
    return jnp.full((16,), s, jnp.float32)


_mesh = plsc.VectorSubcoreMesh(core_axis_name="c", subcore_axis_name="s")


@functools.partial(
    pl.kernel,
    mesh=_mesh,
    compiler_params=pltpu.CompilerParams(needs_layout_passes=False),
    out_type=[
        jax.ShapeDtypeStruct((_B, _V), jnp.float32),
        jax.ShapeDtypeStruct((_B, _SEL), jnp.int32),
        jax.ShapeDtypeStruct((_B, _SEL), jnp.float32),
    ],
    scratch_types=[
        pltpu.VMEM((_NSEG,), jnp.float32),
        pltpu.VMEM((_NSEL,), jnp.int32),
        pltpu.VMEM((_NSEL,), jnp.int32),
        pltpu.VMEM((_NSEL, _SEGW), jnp.float32),
        pltpu.VMEM((_CAP,), jnp.float32),
        pltpu.VMEM((_CAP,), jnp.int32),
        pltpu.VMEM((_SEGW,), jnp.float32),
        pltpu.VMEM((16,), jnp.float32),
        pltpu.VMEM((_V,), jnp.float32),
        pltpu.VMEM((_SEL,), jnp.int32),
        pltpu.VMEM((_SEL,), jnp.float32),
        pltpu.SemaphoreType.DMA,
    ],
)
def _sc_select(u_h, segmin_h, spre_h, masked_h, selidx_h, selval_h,
               segmin_v, segid_v, segbase_v, segbuf, cand_u, cand_i,
               prefix_v, s_v, rowbuf, selidx_v, selval_v, sem):
    wid = lax.axis_index("s") * 2 + lax.axis_index("c")
    lane = lax.broadcasted_iota(jnp.int32, (16,), 0)
    inf_v = jnp.full((16,), jnp.inf, jnp.float32)
    big_v = jnp.full((16,), 2**30, jnp.int32)
    zero_v = jnp.zeros((16,), jnp.float32)

    def zbody(i, carry):
        for t in range(8):
            rowbuf[pl.ds(i * 128 + t * 16, 16)] = zero_v
        return carry

    lax.fori_loop(0, _V // 128, zbody, 0)
    for t in range(_V // 128 * 128, _V, 16):
        rowbuf[pl.ds(t, 16)] = zero_v

    def row_body(rr, carry0):
        row = wid * _RPT + rr
        pltpu.sync_copy(segmin_h.at[row], segmin_v)
        pltpu.sync_copy(spre_h.at[row], s_v)
        pltpu.sync_copy(u_h.at[row, pl.ds(0, _SEGW)], prefix_v)

        nv = _NSEG // 16

        def ext_body(j, st):
            a0, a1, a2, a3, tau = st
            mval = inf_v
            for k in range(nv):
                mval = jnp.minimum(mval, segmin_v[pl.ds(k * 16, 16)])
            mb = _bf(jnp.min(mval))
            mid = big_v
            for k in range(nv):
                v = segmin_v[pl.ds(k * 16, 16)]
                mid = jnp.minimum(
                    mid, jnp.where(v == mb, lane + k * 16, big_v))
            sid = jnp.min(mid)
            sb = _bi(sid)
            pltpu.async_copy(
                u_h.at[row, pl.ds(sid * _SEGW, _SEGW)], segbuf.at[j], sem)
            plsc.store_scatter(segmin_v, [sb], inf_v, mask=lane == 0)
            g = j // 16
            l = j - g * 16
            le = lane == _bi(l)
            gv = _bi(g)
            a0 = jnp.where(le & (gv == 0), sb, a0)
            a1 = jnp.where(le & (gv == 1), sb, a1)
            a2 = jnp.where(le & (gv == 2), sb, a2)
            a3 = jnp.where(le & (gv == 3), sb, a3)
            tau = jnp.where(_bi(j) == _K - 1, mb, tau)
            return (a0, a1, a2, a3, tau)

        a0, a1, a2, a3, tau = lax.fori_loop(
            0, _NSEL, ext_body, (big_v, big_v, big_v, big_v, inf_v))

        avs = (a0, a1, a2, a3)
        for g in range(4):
            segbase_v[pl.ds(g * 16, 16)] = avs[g] * _SEGW
        pltpu.make_async_copy(
            u_h.at[pl.ds(0, _NSEL), pl.ds(0, _SEGW)], segbuf, sem).wait()

        for k in range(_CAP // 16):
            cand_u[pl.ds(k * 16, 16)] = inf_v
            cand_i[pl.ds(k * 16, 16)] = big_v

        def seg_body(s, cnt_v):
            base = plsc.load_gather(segbase_v, [_bi(s)])
            for o in range(_SEGW // 16):
                u_v = segbuf[s, pl.ds(o * 16, 16)]
                m = u_v <= tau
                pc = plsc.all_reduce_population_count(m)
                mi = m.astype(jnp.int32)
                pos = cnt_v + plsc.cumsum(mi) - mi
                wm = m & (pos < _CAP)
                idx_v = base + (o * 16 + lane)
                plsc.store_scatter(cand_u, [pos], u_v, mask=wm)
                plsc.store_scatter(cand_i, [pos], idx_v, mask=wm)
                cnt_v = cnt_v + pc
            return cnt_v

        lax.fori_loop(0, _NSEL, seg_body, jnp.zeros((16,), jnp.int32))

        def fin_body(j, st):
            s0, s1, s2, s3 = st
            mval = inf_v
            for k in range(_CAP // 16):
                mval = jnp.minimum(mval, cand_u[pl.ds(k * 16, 16)])
            mb = _bf(jnp.min(mval))
            mid = big_v
            for k in range(_CAP // 16):
                uv = cand_u[pl.ds(k * 16, 16)]
                iv = cand_i[pl.ds(k * 16, 16)]
                mid = jnp.minimum(mid, jnp.where(uv == mb, iv, big_v))
            ib = _bi(jnp.min(mid))
            for k in range(_CAP // 16):
                uv = cand_u[pl.ds(k * 16, 16)]
                iv = cand_i[pl.ds(k * 16, 16)]
                cand_u[pl.ds(k * 16, 16)] = jnp.where(iv == ib, inf_v, uv)
            g = j // 16
            l = j - g * 16
            le = lane == _bi(l)
            gv = _bi(g)
            s0 = jnp.where(le & (gv == 0), ib, s0)
            s1 = jnp.where(le & (gv == 1), ib, s1)
            s2 = jnp.where(le & (gv == 2), ib, s2)
            s3 = jnp.where(le & (gv == 3), ib, s3)
            return (s0, s1, s2, s3)

        sel = lax.fori_loop(0, _K, fin_body, (big_v, big_v, big_v, big_v))

        sv = s_v[pl.ds(0, 16)]
        for g in range(4):
            selidx_v[pl.ds(g * 16, 16)] = sel[g]
            selval_v[pl.ds(g * 16, 16)] = prefix_v[pl.ds(g * 16, 16)] / sv
        for g in range(4):
            valid = (sel[g] < _bi(_V)) & ((lane + g * 16) < _K)
            pv = selval_v[pl.ds(g * 16, 16)]
            plsc.store_scatter(rowbuf, [sel[g]], pv, mask=valid)
        w0 = rowbuf[pl.ds(0, 16)]
        rowbuf[pl.ds(0, 16)] = jnp.where(lane == 0, zero_v, w0)
        pltpu.sync_copy(rowbuf, masked_h.at[row])
        for g in range(4):
            valid = (sel[g] < _bi(_V)) & ((lane + g * 16) < _K)
            plsc.store_scatter(rowbuf, [sel[g]], zero_v, mask=valid)
        pltpu.sync_copy(selidx_v, selidx_h.at[row])
        pltpu.sync_copy(selval_v, selval_h.at[row])
        return carry0

    lax.fori_loop(0, _RPT, row_body, 0)


def _rotl(x, d):
    return (x << jnp.uint32(d)) | (x >> jnp.uint32(32 - d))


def _sample_body(selidx_ref, selval_ref, out_ref):
    sel = selidx_ref[...]
    val = selval_ref[...]
    col = lax.broadcasted_iota(jnp.int32, (_B, _SEL), 1)
    row = lax.broadcasted_iota(jnp.int32, (_B, _SEL), 0)
    valid = (col < _K) & (sel != 0) & (sel < _V)
    x1 = (row * _V + sel).astype(jnp.uint32)
    x0 = jnp.zeros((_B, _SEL), jnp.uint32)
    ks = (jnp.uint32(0), jnp.uint32(1),
          jnp.uint32(0x1BD11BDA) ^ jnp.uint32(0) ^ jnp.uint32(1))
    x0 = x0 + ks[0]
    x1 = x1 + ks[1]
    rotations = ((13, 15, 26, 6), (17, 29, 16, 24))
    for i in range(5):
        for r in rotations[i % 2]:
            x0 = x0 + x1
            x1 = _rotl(x1, r) ^ x0
        x0 = x0 + ks[(i + 1) % 3]
        x1 = x1 + ks[(i + 2) % 3] + jnp.uint32(i + 1)
    bits = x0 ^ x1
    fb = (bits >> jnp.uint32(9)) | jnp.uint32(0x3F800000)
    tiny = jnp.float32(jnp.finfo(jnp.float32).tiny)
    fl = lax.bitcast_convert_type(fb, jnp.float32) - 1.0
    un = jnp.maximum(tiny, fl * (jnp.float32(1.0) - tiny) + tiny)
    g = -jnp.log(-jnp.log(un))
    score = jnp.where(valid, jnp.log(jnp.maximum(val, 1e-30)) + g,
                      -jnp.float32(jnp.inf))
    best = jnp.max(score, axis=1, keepdims=True)
    samp = jnp.min(jnp.where(score == best, sel, jnp.int32(_V + 1)),
                   axis=1, keepdims=True)
    out_ref[...] = samp


def kernel(logits):
    nblk = _B // _RB
    ymax = pl.pallas_call(
        _k1a_body,
        grid=(nblk, _NCHUNKA),
        in_specs=[pl.BlockSpec((_RB, _CHUNKA), lambda i, c: (i, c))],
        out_specs=pl.BlockSpec((_RB, 128), lambda i, c: (i, 0)),
        out_shape=jax.ShapeDtypeStruct((_B, 128), jnp.float32),
    )(logits)

    u, segmin, s = pl.pallas_call(
        _k1b_body,
        grid=(nblk, _NCHUNK),
        in_specs=[
            pl.BlockSpec((_RB, _CHUNK), lambda i, c: (i, c)),
            pl.BlockSpec((_RB, 128), lambda i, c: (i, 0)),
        ],
        out_specs=[
            pl.BlockSpec((_RB, _CHUNK), lambda i, c: (i, c)),
            pl.BlockSpec((_RB, _NSEG), lambda i, c: (i, 0)),
            pl.BlockSpec((_RB, 128), lambda i, c: (i, 0)),
        ],
        out_shape=[
            jax.ShapeDtypeStruct((_B, _VP), jnp.float32),
            jax.ShapeDtypeStruct((_B, _NSEG), jnp.float32),
            jax.ShapeDtypeStruct((_B, 128), jnp.float32),
        ],
    )(logits, ymax)

    spre = s[:, :16]
    masked, selidx, selval = _sc_select(u, segmin, spre)

    sample = pl.pallas_call(
        _sample_body,
        out_shape=jax.ShapeDtypeStruct((_B, 1), jnp.int32),
    )(selidx, selval)
    return masked, sample.astype(jnp.int64)


# speedup vs baseline: 25.8480x; 1.0414x over previous
import functools

import jax
import jax.numpy as jnp
from jax import lax
from jax.experimental import pallas as pl
from jax.experimental.pallas import tpu as pltpu
from jax.experimental.pallas import tpu_sc as plsc

_TEMP = 0.7
_K = 50
_B = 128
_V = 100000
_SEGW = 256
_NSEG = 512
_VP = 131072
_CHUNK = 32768
_NCHUNK = 4
_CHUNKA = 32768
_NCHUNKA = 4
_RB = 8
_NSEL = 64
_CAP = 256
_RPT = 4
_SEL = 64


def _k1a_body(x_ref, ymax_ref):
    c = pl.program_id(1)
    x = x_ref[...]
    last = _NCHUNKA - 1

    @pl.when(c == 0)
    def _():
        ymax_ref[...] = jnp.broadcast_to(
            jnp.max(x, axis=1, keepdims=True), (_RB, 128))

    @pl.when((c > 0) & (c < last))
    def _():
        mb = jnp.broadcast_to(jnp.max(x, axis=1, keepdims=True), (_RB, 128))
        ymax_ref[...] = jnp.maximum(ymax_ref[...], mb)

    @pl.when(c == last)
    def _():
        col = c * _CHUNKA + lax.broadcasted_iota(
            jnp.int32, (_RB, _CHUNKA), 1)
        xm = jnp.where(col < _V, x, -jnp.float32(jnp.inf))
        mb = jnp.broadcast_to(jnp.max(xm, axis=1, keepdims=True), (_RB, 128))
        ymax_ref[...] = jnp.maximum(ymax_ref[...], mb)


def _k1b_tail(c, u_out, s_part, u_ref, segmin_ref, s_ref):
    nsegs = _CHUNK // _SEGW
    u_ref[...] = u_out
    mins = [jnp.min(u_out[:, t * _SEGW:(t + 1) * _SEGW], axis=1,
                    keepdims=True) for t in range(nsegs)]
    segmin_ref[:, pl.ds(c * nsegs, nsegs)] = jnp.concatenate(mins, axis=1)
    sb = jnp.broadcast_to(s_part, (_RB, 128))

    @pl.when(c == 0)
    def _():
        s_ref[...] = sb

    @pl.when(c != 0)
    def _():
        s_ref[...] = s_ref[...] + sb


def _k1b_body(x_ref, ymax_ref, u_ref, segmin_ref, s_ref):
    c = pl.program_id(1)
    x = x_ref[...]
    ym = ymax_ref[...][:, :1] / jnp.float32(_TEMP)
    last = _NCHUNK - 1

    @pl.when(c < last)
    def _():
        u = jnp.exp(x / jnp.float32(_TEMP) - ym)
        _k1b_tail(c, u, jnp.sum(u, axis=1, keepdims=True),
                  u_ref, segmin_ref, s_ref)

    @pl.when(c == last)
    def _():
        col = c * _CHUNK + lax.broadcasted_iota(jnp.int32, (_RB, _CHUNK), 1)
        valid = col < _V
        u = jnp.exp(x / jnp.float32(_TEMP) - ym)
        u_out = jnp.where(valid, u, jnp.float32(jnp.inf))
        s_part = jnp.sum(jnp.where(valid, u, 0.0), axis=1, keepdims=True)
        _k1b_tail(c, u_out, s_part, u_ref, segmin_ref, s_ref)


def _bi(s):
    return jnp.full((16,), s, jnp.int32)


def _bf(s):
    return jnp.full((16,), s, jnp.float32)


_mesh = plsc.VectorSubcoreMesh(core_axis_name="c", subcore_axis_name="s")


@functools.partial(
    pl.kernel,
    mesh=_mesh,
    compiler_params=pltpu.CompilerParams(needs_layout_passes=False),
    out_type=[
        jax.ShapeDtypeStruct((_B, _V), jnp.float32),
        jax.ShapeDtypeStruct((_B, _SEL), jnp.int32),
        jax.ShapeDtypeStruct((_B, _SEL), jnp.float32),
    ],
    scratch_types=[
        pltpu.VMEM((_NSEG,), jnp.float32),
        pltpu.VMEM((_NSEL,), jnp.int32),
        pltpu.VMEM((_NSEL,), jnp.int32),
        pltpu.VMEM((_NSEL, _SEGW), jnp.float32),
        pltpu.VMEM((_CAP,), jnp.float32),
        pltpu.VMEM((_CAP,), jnp.int32),
        pltpu.VMEM((_SEGW,), jnp.float32),
        pltpu.VMEM((16,), jnp.float32),
        pltpu.VMEM((_V,), jnp.float32),
        pltpu.VMEM((_SEL,), jnp.int32),
        pltpu.VMEM((_SEL,), jnp.float32),
        pltpu.SemaphoreType.DMA,
    ],
)
def _sc_select(u_h, segmin_h, spre_h, masked_h, selidx_h, selval_h,
               segmin_v, segid_v, segbase_v, segbuf, cand_u, cand_i,
               prefix_v, s_v, rowbuf, selidx_v, selval_v, sem):
    wid = lax.axis_index("s") * 2 + lax.axis_index("c")
    lane = lax.broadcasted_iota(jnp.int32, (16,), 0)
    inf_v = jnp.full((16,), jnp.inf, jnp.float32)
    big_v = jnp.full((16,), 2**30, jnp.int32)
    zero_v = jnp.zeros((16,), jnp.float32)

    def zbody(i, carry):
        for t in range(8):
            rowbuf[pl.ds(i * 128 + t * 16, 16)] = zero_v
        return carry

    lax.fori_loop(0, _V // 128, zbody, 0)
    for t in range(_V // 128 * 128, _V, 16):
        rowbuf[pl.ds(t, 16)] = zero_v

    def row_body(rr, carry0):
        row = wid * _RPT + rr
        pltpu.sync_copy(segmin_h.at[row], segmin_v)
        pltpu.sync_copy(spre_h.at[row], s_v)
        pltpu.sync_copy(u_h.at[row, pl.ds(0, _SEGW)], prefix_v)

        nv = _NSEG // 16

        def ext_body(j, st):
            a0, a1, a2, a3, tau = st
            mval = inf_v
            for k in range(nv):
                mval = jnp.minimum(mval, segmin_v[pl.ds(k * 16, 16)])
            mb = _bf(jnp.min(mval))
            mid = big_v
            for k in range(nv):
                v = segmin_v[pl.ds(k * 16, 16)]
                mid = jnp.minimum(
                    mid, jnp.where(v == mb, lane + k * 16, big_v))
            sid = jnp.min(mid)
            sb = _bi(sid)
            pltpu.async_copy(
                u_h.at[row, pl.ds(sid * _SEGW, _SEGW)], segbuf.at[j], sem)
            plsc.store_scatter(segmin_v, [sb], inf_v, mask=lane == 0)
            g = j // 16
            l = j - g * 16
            le = lane == _bi(l)
            gv = _bi(g)
            a0 = jnp.where(le & (gv == 0), sb, a0)
            a1 = jnp.where(le & (gv == 1), sb, a1)
            a2 = jnp.where(le & (gv == 2), sb, a2)
            a3 = jnp.where(le & (gv == 3), sb, a3)
            tau = jnp.where(_bi(j) == _K - 1, mb, tau)
            return (a0, a1, a2, a3, tau)

        a0, a1, a2, a3, tau = lax.fori_loop(
            0, _NSEL, ext_body, (big_v, big_v, big_v, big_v, inf_v))

        avs = (a0, a1, a2, a3)
        for g in range(4):
            segbase_v[pl.ds(g * 16, 16)] = avs[g] * _SEGW
        pltpu.make_async_copy(
            u_h.at[pl.ds(0, _NSEL), pl.ds(0, _SEGW)], segbuf, sem).wait()

        for k in range(_CAP // 16):
            cand_u[pl.ds(k * 16, 16)] = inf_v
            cand_i[pl.ds(k * 16, 16)] = big_v

        def seg_body(s, cnt_v):
            base = plsc.load_gather(segbase_v, [_bi(s)])
            for o in range(_SEGW // 16):
                u_v = segbuf[s, pl.ds(o * 16, 16)]
                m = u_v <= tau
                pc = plsc.all_reduce_population_count(m)
                mi = m.astype(jnp.int32)
                pos = cnt_v + plsc.cumsum(mi) - mi
                wm = m & (pos < _CAP)
                idx_v = base + (o * 16 + lane)
                plsc.store_scatter(cand_u, [pos], u_v, mask=wm)
                plsc.store_scatter(cand_i, [pos], idx_v, mask=wm)
                cnt_v = cnt_v + pc
            return cnt_v

        lax.fori_loop(0, _NSEL, seg_body, jnp.zeros((16,), jnp.int32))

        def fin_body(j, st):
            s0, s1, s2, s3 = st
            mval = inf_v
            for k in range(_CAP // 16):
                mval = jnp.minimum(mval, cand_u[pl.ds(k * 16, 16)])
            mb = _bf(jnp.min(mval))
            mid = big_v
            for k in range(_CAP // 16):
                uv = cand_u[pl.ds(k * 16, 16)]
                iv = cand_i[pl.ds(k * 16, 16)]
                mid = jnp.minimum(mid, jnp.where(uv == mb, iv, big_v))
            ib = _bi(jnp.min(mid))
            for k in range(_CAP // 16):
                uv = cand_u[pl.ds(k * 16, 16)]
                iv = cand_i[pl.ds(k * 16, 16)]
                cand_u[pl.ds(k * 16, 16)] = jnp.where(iv == ib, inf_v, uv)
            g = j // 16
            l = j - g * 16
            le = lane == _bi(l)
            gv = _bi(g)
            s0 = jnp.where(le & (gv == 0), ib, s0)
            s1 = jnp.where(le & (gv == 1), ib, s1)
            s2 = jnp.where(le & (gv == 2), ib, s2)
            s3 = jnp.where(le & (gv == 3), ib, s3)
            return (s0, s1, s2, s3)

        sel = lax.fori_loop(0, _K, fin_body, (big_v, big_v, big_v, big_v))

        sv = s_v[pl.ds(0, 16)]
        for g in range(4):
            selidx_v[pl.ds(g * 16, 16)] = sel[g]
            selval_v[pl.ds(g * 16, 16)] = prefix_v[pl.ds(g * 16, 16)] / sv
        for g in range(4):
            valid = (sel[g] < _bi(_V)) & ((lane + g * 16) < _K)
            pv = selval_v[pl.ds(g * 16, 16)]
            plsc.store_scatter(rowbuf, [sel[g]], pv, mask=valid)
        w0 = rowbuf[pl.ds(0, 16)]
        rowbuf[pl.ds(0, 16)] = jnp.where(lane == 0, zero_v, w0)
        pltpu.sync_copy(rowbuf, masked_h.at[row])
        for g in range(4):
            valid = (sel[g] < _bi(_V)) & ((lane + g * 16) < _K)
            plsc.store_scatter(rowbuf, [sel[g]], zero_v, mask=valid)
        pltpu.sync_copy(selidx_v, selidx_h.at[row])
        pltpu.sync_copy(selval_v, selval_h.at[row])
        return carry0

    lax.fori_loop(0, _RPT, row_body, 0)


def _rotl(x, d):
    return (x << jnp.uint32(d)) | (x >> jnp.uint32(32 - d))


def _sample_body(selidx_ref, selval_ref, out_ref):
    sel = selidx_ref[...]
    val = selval_ref[...]
    col = lax.broadcasted_iota(jnp.int32, (_B, _SEL), 1)
    row = lax.broadcasted_iota(jnp.int32, (_B, _SEL), 0)
    valid = (col < _K) & (sel != 0) & (sel < _V)
    x1 = (row * _V + sel).astype(jnp.uint32)
    x0 = jnp.zeros((_B, _SEL), jnp.uint32)
    ks = (jnp.uint32(0), jnp.uint32(1),
          jnp.uint32(0x1BD11BDA) ^ jnp.uint32(0) ^ jnp.uint32(1))
    x0 = x0 + ks[0]
    x1 = x1 + ks[1]
    rotations = ((13, 15, 26, 6), (17, 29, 16, 24))
    for i in range(5):
        for r in rotations[i % 2]:
            x0 = x0 + x1
            x1 = _rotl(x1, r) ^ x0
        x0 = x0 + ks[(i + 1) % 3]
        x1 = x1 + ks[(i + 2) % 3] + jnp.uint32(i + 1)
    bits = x0 ^ x1
    fb = (bits >> jnp.uint32(9)) | jnp.uint32(0x3F800000)
    tiny = jnp.float32(jnp.finfo(jnp.float32).tiny)
    fl = lax.bitcast_convert_type(fb, jnp.float32) - 1.0
    un = jnp.maximum(tiny, fl * (jnp.float32(1.0) - tiny) + tiny)
    g = -jnp.log(-jnp.log(un))
    score = jnp.where(valid, jnp.log(jnp.maximum(val, 1e-30)) + g,
                      -jnp.float32(jnp.inf))
    best = jnp.max(score, axis=1, keepdims=True)
    samp = jnp.min(jnp.where(score == best, sel, jnp.int32(_V + 1)),
                   axis=1, keepdims=True)
    out_ref[...] = samp


def kernel(logits):
    nblk = _B // _RB
    ymax = pl.pallas_call(
        _k1a_body,
        grid=(nblk, _NCHUNKA),
        in_specs=[pl.BlockSpec((_RB, _CHUNKA), lambda i, c: (i, c))],
        out_specs=pl.BlockSpec((_RB, 128), lambda i, c: (i, 0)),
        out_shape=jax.ShapeDtypeStruct((_B, 128), jnp.float32),
    )(logits)

    u, segmin, s = pl.pallas_call(
        _k1b_body,
        grid=(nblk, _NCHUNK),
        in_specs=[
            pl.BlockSpec((_RB, _CHUNK), lambda i, c: (i, c)),
            pl.BlockSpec((_RB, 128), lambda i, c: (i, 0)),
        ],
        out_specs=[
            pl.BlockSpec((_RB, _CHUNK), lambda i, c: (i, c)),
            pl.BlockSpec((_RB, _NSEG), lambda i, c: (i, 0)),
            pl.BlockSpec((_RB, 128), lambda i, c: (i, 0)),
        ],
        out_shape=[
            jax.ShapeDtypeStruct((_B, _VP), jnp.float32),
            jax.ShapeDtypeStruct((_B, _NSEG), jnp.float32),
            jax.ShapeDtypeStruct((_B, 128), jnp.float32),
        ],
    )(logits, ymax)

    spre = s[:, :16]
    masked, selidx, selval = _sc_select(u, segmin, spre)

    sample = pl.pallas_call(
        _sample_body,
        out_shape=jax.ShapeDtypeStruct((_B, 1), jnp.int32),
    )(selidx, selval)
    return masked, sample.astype(jnp.int64)
